# Initial kernel scaffold; baseline (speedup 1.0000x reference)
import jax
import jax.numpy as jnp
from jax.experimental import pallas as pl


def kernel(x, edge_index, edge_values, W1, b1, W2, b2, W3, b3, Wlin, blin):
    raise NotImplementedError("write your pallas kernel here")


# speedup vs baseline: 3.7994x; 3.7994x over previous
import functools

import jax
import jax.numpy as jnp
from jax import lax
from jax.experimental import pallas as pl
from jax.experimental.pallas import tpu as pltpu
from jax.experimental.pallas import tpu_sc as plsc

_N = 10000
_E = 320000
_F = 128
_NCLASS = 40

_NW = 32
_CHUNK = 128
_EPW = 10240
_EP = _NW * _EPW
_NCHUNKS = _EPW // _CHUNK
_NPAD = 10240
_RPT = _NPAD // 16

_BN = 1000


def _mm_body(x_ref, w_ref, o_ref):
    o_ref[...] = jnp.dot(x_ref[...], w_ref[...],
                         preferred_element_type=jnp.float32)


def _fuse_body(p_ref, b_ref, w_ref, x_ref, hw_ref):
    xb = jnp.maximum(p_ref[0] + p_ref[1] + b_ref[...], 0.0)
    x_ref[...] = xb
    hw_ref[...] = jnp.dot(xb, w_ref[...], preferred_element_type=jnp.float32)


def _final_body(p_ref, b_ref, x1_ref, x2_ref, wl1_ref, wl2_ref, wl3_ref,
                bl_ref, o_ref):
    x3 = jnp.maximum(p_ref[0] + p_ref[1] + b_ref[...], 0.0)
    logits = (jnp.dot(x1_ref[...], wl1_ref[...],
                      preferred_element_type=jnp.float32)
              + jnp.dot(x2_ref[...], wl2_ref[...],
                        preferred_element_type=jnp.float32)
              + jnp.dot(x3, wl3_ref[...], preferred_element_type=jnp.float32)
              + bl_ref[...])
    m = jnp.max(logits, axis=1, keepdims=True)
    s = logits - m
    o_ref[...] = s - jnp.log(jnp.sum(jnp.exp(s), axis=1, keepdims=True))


def _mm(x, w):
    return pl.pallas_call(
        _mm_body,
        grid=(_N // _BN,),
        in_specs=[pl.BlockSpec((_BN, _F), lambda i: (i, 0)),
                  pl.BlockSpec((_F, _F), lambda i: (0, 0))],
        out_specs=pl.BlockSpec((_BN, _F), lambda i: (i, 0)),
        out_shape=jax.ShapeDtypeStruct((_N, _F), jnp.float32),
    )(x, w)


def _fuse(p, b2d, w):
    return pl.pallas_call(
        _fuse_body,
        grid=(_N // _BN,),
        in_specs=[pl.BlockSpec((2, _BN, _F), lambda i: (0, i, 0)),
                  pl.BlockSpec((1, _F), lambda i: (0, 0)),
                  pl.BlockSpec((_F, _F), lambda i: (0, 0))],
        out_specs=[pl.BlockSpec((_BN, _F), lambda i: (i, 0)),
                   pl.BlockSpec((_BN, _F), lambda i: (i, 0))],
        out_shape=[jax.ShapeDtypeStruct((_N, _F), jnp.float32),
                   jax.ShapeDtypeStruct((_N, _F), jnp.float32)],
    )(p, b2d, w)


def _final(p, b2d, x1, x2, wl1, wl2, wl3, bl2d):
    return pl.pallas_call(
        _final_body,
        grid=(_N // _BN,),
        in_specs=[pl.BlockSpec((2, _BN, _F), lambda i: (0, i, 0)),
                  pl.BlockSpec((1, _F), lambda i: (0, 0)),
                  pl.BlockSpec((_BN, _F), lambda i: (i, 0)),
                  pl.BlockSpec((_BN, _F), lambda i: (i, 0)),
                  pl.BlockSpec((_F, _NCLASS), lambda i: (0, 0)),
                  pl.BlockSpec((_F, _NCLASS), lambda i: (0, 0)),
                  pl.BlockSpec((_F, _NCLASS), lambda i: (0, 0)),
                  pl.BlockSpec((1, _NCLASS), lambda i: (0, 0))],
        out_specs=pl.BlockSpec((_BN, _NCLASS), lambda i: (i, 0)),
        out_shape=jax.ShapeDtypeStruct((_N, _NCLASS), jnp.float32),
    )(p, b2d, x1, x2, wl1, wl2, wl3, bl2d)


_BLK = 8


def _edge_body(hw_hbm, src_hbm, dst_hbm, ev_hbm, out_hbm,
               acc_sh, sblk, dblk, eblk, rows_v, sb, sg, ss):
    cid = lax.axis_index("c")
    sid = lax.axis_index("s")
    wid = cid * 16 + sid
    wrow = wid * _NCHUNKS
    nblk = _NCHUNKS // _BLK

    z = jnp.zeros((16,), jnp.float32)

    def zrow(r, carry):
        for j in range(8):
            rows_v[0, r, pl.ds(j * 16, 16)] = z
        return carry

    lax.fori_loop(0, _CHUNK, zrow, 0)
    for q in range(_RPT // _CHUNK):
        pltpu.sync_copy(
            rows_v.at[0], acc_sh.at[pl.ds(sid * _RPT + q * _CHUNK, _CHUNK)])
    plsc.subcore_barrier()

    def blk_copies(to):
        rs = pl.ds(wrow + to * _BLK, _BLK)
        bslot = lax.rem(to, 3)
        sem = sb.at[bslot]
        return (pltpu.make_async_copy(src_hbm.at[rs], sblk.at[bslot], sem),
                pltpu.make_async_copy(dst_hbm.at[rs], dblk.at[bslot], sem),
                pltpu.make_async_copy(ev_hbm.at[rs], eblk.at[bslot], sem))

    def gather_desc(t):
        bslot = lax.rem(lax.div(t, _BLK), 3)
        row = lax.rem(t, _BLK)
        slot = lax.rem(t, 2)
        return pltpu.make_async_copy(hw_hbm.at[sblk.at[bslot, row]],
                                     rows_v.at[slot], sg.at[slot])

    def scatter_desc(t):
        bslot = lax.rem(lax.div(t, _BLK), 3)
        row = lax.rem(t, _BLK)
        slot = lax.rem(t, 2)
        return pltpu.make_async_copy(rows_v.at[slot],
                                     acc_sh.at[dblk.at[bslot, row]],
                                     ss.at[slot])

    for c in blk_copies(0):
        c.start()
    for c in blk_copies(0):
        c.wait()
    gather_desc(0).start()
    for c in blk_copies(1):
        c.start()

    def step(t, carry):
        bslot = lax.rem(lax.div(t, _BLK), 3)
        bi = lax.rem(t, _BLK)
        slot = lax.rem(t, 2)
        to = lax.div(t, _BLK)
        last_in_blk = bi == _BLK - 1

        @pl.when(jnp.logical_and(last_in_blk, t + 1 < _NCHUNKS))
        def _():
            for c in blk_copies(to + 1):
                c.wait()

        @pl.when(t >= 1)
        def _():
            scatter_desc(t - 1).wait()

        @pl.when(t + 1 < _NCHUNKS)
        def _():
            gather_desc(t + 1).start()

        gather_desc(t).wait()

        def group(g, c2):
            ev16 = eblk[bslot, bi, pl.ds(g * 16, 16)]
            for e in range(16):
                w = lax.gather(
                    ev16, jnp.full((16, 1), e, jnp.int32),
                    dimension_numbers=lax.GatherDimensionNumbers(
                        offset_dims=(), collapsed_slice_dims=(0,),
                        start_index_map=(0,)),
                    slice_sizes=(1,),
                    mode=lax.GatherScatterMode.PROMISE_IN_BOUNDS)
                k = g * 16 + e
                for j in range(8):
                    rows_v[slot, k, pl.ds(j * 16, 16)] = (
                        rows_v[slot, k, pl.ds(j * 16, 16)] * w)
            return c2

        lax.fori_loop(0, _CHUNK // 16, group, 0)

        pltpu.async_copy(rows_v.at[slot],
                         acc_sh.at[dblk.at[bslot, bi]],
                         ss.at[slot], add=True)

        @pl.when(jnp.logical_and(last_in_blk, to + 2 < nblk))
        def _():
            for c in blk_copies(to + 2):
                c.start()

        return carry

    lax.fori_loop(0, _NCHUNKS, step, 0)
    scatter_desc(_NCHUNKS - 1).wait()
    plsc.subcore_barrier()
    pltpu.sync_copy(acc_sh.at[pl.ds(sid * _RPT, _RPT)],
                    out_hbm.at[cid, pl.ds(sid * _RPT, _RPT)])


@functools.cache
def _make_edge_pass():
    mesh = plsc.VectorSubcoreMesh(core_axis_name="c", subcore_axis_name="s")
    return pl.kernel(
        _edge_body,
        out_type=jax.ShapeDtypeStruct((2, _NPAD, _F), jnp.float32),
        mesh=mesh,
        scratch_types=[
            pltpu.VMEM_SHARED((_NPAD, _F), jnp.float32),
            pltpu.VMEM((3, _BLK, _CHUNK), jnp.int32),
            pltpu.VMEM((3, _BLK, _CHUNK), jnp.int32),
            pltpu.VMEM((3, _BLK, _CHUNK), jnp.float32),
            pltpu.VMEM((2, _CHUNK, _F), jnp.float32),
            pltpu.SemaphoreType.DMA((3,)),
            pltpu.SemaphoreType.DMA((2,)),
            pltpu.SemaphoreType.DMA((2,)),
        ],
    )


def kernel(x, edge_index, edge_values, W1, b1, W2, b2, W3, b3, Wlin, blin):
    src = edge_index[0]
    dst = edge_index[1]

    pad = _EP - _E
    fill = jnp.arange(pad, dtype=jnp.int32) % _N
    rows = _NW * _NCHUNKS
    srcp = jnp.pad(jnp.concatenate([src, fill]).reshape(rows, _CHUNK),
                   ((0, _BLK), (0, 0)))
    dstp = jnp.pad(jnp.concatenate([dst, fill]).reshape(rows, _CHUNK),
                   ((0, _BLK), (0, 0)))
    evp = jnp.pad(
        jnp.concatenate([edge_values,
                         jnp.zeros((pad,), jnp.float32)]).reshape(rows, _CHUNK),
        ((0, _BLK), (0, 0)))

    edge_pass = _make_edge_pass()

    hw1 = _mm(x, W1)
    p1 = edge_pass(hw1, srcp, dstp, evp)
    x1, hw2 = _fuse(p1, b1.reshape(1, -1), W2)
    p2 = edge_pass(hw2, srcp, dstp, evp)
    x2, hw3 = _fuse(p2, b2.reshape(1, -1), W3)
    p3 = edge_pass(hw3, srcp, dstp, evp)
    out = _final(p3, b3.reshape(1, -1), x1, x2,
                 Wlin[0:_F], Wlin[_F:2 * _F], Wlin[2 * _F:3 * _F],
                 blin.reshape(1, -1))
    return out


# speedup vs baseline: 11.2067x; 2.9496x over previous
import functools

import jax
import jax.numpy as jnp
from jax import lax
from jax.experimental import pallas as pl
from jax.experimental.pallas import tpu as pltpu
from jax.experimental.pallas import tpu_sc as plsc

_N = 10000
_E = 320000
_F = 128
_NCLASS = 40

_NW = 32
_CHUNK = 128
_EPW = 10240
_EP = _NW * _EPW
_NCHUNKS = _EPW // _CHUNK
_NPAD = 10240
_RPT = _NPAD // 16

_BN = 1000


def _mm_body(x_ref, w_ref, o_ref):
    o_ref[...] = jnp.dot(x_ref[...], w_ref[...],
                         preferred_element_type=jnp.float32)


def _fuse_body(p_ref, b_ref, w_ref, x_ref, hw_ref):
    xb = jnp.maximum(p_ref[0] + p_ref[1] + b_ref[...], 0.0)
    x_ref[...] = xb
    hw_ref[...] = jnp.dot(xb, w_ref[...], preferred_element_type=jnp.float32)


def _final_body(p_ref, b_ref, x1_ref, x2_ref, wl1_ref, wl2_ref, wl3_ref,
                bl_ref, o_ref):
    x3 = jnp.maximum(p_ref[0] + p_ref[1] + b_ref[...], 0.0)
    logits = (jnp.dot(x1_ref[...], wl1_ref[...],
                      preferred_element_type=jnp.float32)
              + jnp.dot(x2_ref[...], wl2_ref[...],
                        preferred_element_type=jnp.float32)
              + jnp.dot(x3, wl3_ref[...], preferred_element_type=jnp.float32)
              + bl_ref[...])
    m = jnp.max(logits, axis=1, keepdims=True)
    s = logits - m
    o_ref[...] = s - jnp.log(jnp.sum(jnp.exp(s), axis=1, keepdims=True))


def _mm(x, w):
    return pl.pallas_call(
        _mm_body,
        grid=(_N // _BN,),
        in_specs=[pl.BlockSpec((_BN, _F), lambda i: (i, 0)),
                  pl.BlockSpec((_F, _F), lambda i: (0, 0))],
        out_specs=pl.BlockSpec((_BN, _F), lambda i: (i, 0)),
        out_shape=jax.ShapeDtypeStruct((_N, _F), jnp.float32),
    )(x, w)


def _fuse(p, b2d, w):
    return pl.pallas_call(
        _fuse_body,
        grid=(_N // _BN,),
        in_specs=[pl.BlockSpec((2, _BN, _F), lambda i: (0, i, 0)),
                  pl.BlockSpec((1, _F), lambda i: (0, 0)),
                  pl.BlockSpec((_F, _F), lambda i: (0, 0))],
        out_specs=[pl.BlockSpec((_BN, _F), lambda i: (i, 0)),
                   pl.BlockSpec((_BN, _F), lambda i: (i, 0))],
        out_shape=[jax.ShapeDtypeStruct((_N, _F), jnp.float32),
                   jax.ShapeDtypeStruct((_N, _F), jnp.float32)],
    )(p, b2d, w)


def _final(p, b2d, x1, x2, wl1, wl2, wl3, bl2d):
    return pl.pallas_call(
        _final_body,
        grid=(_N // _BN,),
        in_specs=[pl.BlockSpec((2, _BN, _F), lambda i: (0, i, 0)),
                  pl.BlockSpec((1, _F), lambda i: (0, 0)),
                  pl.BlockSpec((_BN, _F), lambda i: (i, 0)),
                  pl.BlockSpec((_BN, _F), lambda i: (i, 0)),
                  pl.BlockSpec((_F, _NCLASS), lambda i: (0, 0)),
                  pl.BlockSpec((_F, _NCLASS), lambda i: (0, 0)),
                  pl.BlockSpec((_F, _NCLASS), lambda i: (0, 0)),
                  pl.BlockSpec((1, _NCLASS), lambda i: (0, 0))],
        out_specs=pl.BlockSpec((_BN, _NCLASS), lambda i: (i, 0)),
        out_shape=jax.ShapeDtypeStruct((_N, _NCLASS), jnp.float32),
    )(p, b2d, x1, x2, wl1, wl2, wl3, bl2d)


_BLK = 8


def _edge_body(hw_hbm, src_hbm, dst_hbm, ev_hbm, out_hbm,
               acc_sh, sblk, dblk, eblk, rows_v, sb, sg, ss):
    cid = lax.axis_index("c")
    sid = lax.axis_index("s")
    wid = cid * 16 + sid
    wrow = wid * _NCHUNKS
    nblk = _NCHUNKS // _BLK

    z = jnp.zeros((16,), jnp.float32)

    def zrow(r, carry):
        for j in range(8):
            rows_v[0, r, pl.ds(j * 16, 16)] = z
        return carry

    lax.fori_loop(0, _CHUNK, zrow, 0)
    for q in range(_RPT // _CHUNK):
        pltpu.sync_copy(
            rows_v.at[0], acc_sh.at[pl.ds(sid * _RPT + q * _CHUNK, _CHUNK)])
    plsc.subcore_barrier()

    def blk_copies(to):
        rs = pl.ds(wrow + to * _BLK, _BLK)
        bslot = lax.rem(to, 3)
        sem = sb.at[bslot]
        return (pltpu.make_async_copy(src_hbm.at[rs], sblk.at[bslot], sem),
                pltpu.make_async_copy(dst_hbm.at[rs], dblk.at[bslot], sem),
                pltpu.make_async_copy(ev_hbm.at[rs], eblk.at[bslot], sem))

    def gather_desc(t):
        bslot = lax.rem(lax.div(t, _BLK), 3)
        row = lax.rem(t, _BLK)
        slot = lax.rem(t, 2)
        return pltpu.make_async_copy(hw_hbm.at[sblk.at[bslot, row]],
                                     rows_v.at[slot], sg.at[slot])

    def scatter_desc(t):
        bslot = lax.rem(lax.div(t, _BLK), 3)
        row = lax.rem(t, _BLK)
        slot = lax.rem(t, 2)
        return pltpu.make_async_copy(rows_v.at[slot],
                                     acc_sh.at[dblk.at[bslot, row]],
                                     ss.at[slot])

    def scale_rows(slot, bslot, bi):
        def group(g, c2):
            ev16 = eblk[bslot, bi, pl.ds(g * 16, 16)]
            for e in range(16):
                w = lax.gather(
                    ev16, jnp.full((16, 1), e, jnp.int32),
                    dimension_numbers=lax.GatherDimensionNumbers(
                        offset_dims=(), collapsed_slice_dims=(0,),
                        start_index_map=(0,)),
                    slice_sizes=(1,),
                    mode=lax.GatherScatterMode.PROMISE_IN_BOUNDS)
                k = g * 16 + e
                for j in range(8):
                    rows_v[slot, k, pl.ds(j * 16, 16)] = (
                        rows_v[slot, k, pl.ds(j * 16, 16)] * w)
            return c2

        lax.fori_loop(0, _CHUNK // 16, group, 0)

    def g_start(t, slot):
        bslot = lax.rem(lax.div(t, _BLK), 3)
        pltpu.async_copy(hw_hbm.at[sblk.at[bslot, lax.rem(t, _BLK)]],
                         rows_v.at[slot], sg.at[slot])

    def g_wait(slot):
        pltpu.make_async_copy(hw_hbm.at[sblk.at[0, 0]],
                              rows_v.at[slot], sg.at[slot]).wait()

    def s_start(t, slot, bslot, bi):
        pltpu.async_copy(rows_v.at[slot], acc_sh.at[dblk.at[bslot, bi]],
                         ss.at[slot], add=True)

    def s_wait(slot):
        pltpu.make_async_copy(rows_v.at[slot], acc_sh.at[dblk.at[0, 0]],
                              ss.at[slot]).wait()

    for c in blk_copies(0):
        c.start()
    for c in blk_copies(0):
        c.wait()
    g_start(0, 0)

    def outer(to, carry):
        bslot = lax.rem(to, 3)

        @pl.when(to < nblk - 1)
        def _():
            for c in blk_copies(to + 1):
                c.start()

        def inner(ti, c2):
            for u in (0, 1):
                t = to * _BLK + 2 * ti + u
                bi = 2 * ti + u
                if u == 0:
                    @pl.when(t >= 1)
                    def _():
                        s_wait(1)
                    g_start(t + 1, 1)
                else:
                    s_wait(0)

                    @pl.when(t + 1 < _NCHUNKS)
                    def _():
                        @pl.when(ti == _BLK // 2 - 1)
                        def _():
                            for c in blk_copies(to + 1):
                                c.wait()
                        g_start(t + 1, 0)
                g_wait(u)
                scale_rows(u, bslot, bi)
                s_start(t, u, bslot, bi)
            return c2

        lax.fori_loop(0, _BLK // 2, inner, 0)
        return carry

    lax.fori_loop(0, nblk, outer, 0)
    s_wait(1)
    plsc.subcore_barrier()
    pltpu.sync_copy(acc_sh.at[pl.ds(sid * _RPT, _RPT)],
                    out_hbm.at[cid, pl.ds(sid * _RPT, _RPT)])


@functools.cache
def _make_edge_pass():
    mesh = plsc.VectorSubcoreMesh(core_axis_name="c", subcore_axis_name="s")
    return pl.kernel(
        _edge_body,
        out_type=jax.ShapeDtypeStruct((2, _NPAD, _F), jnp.float32),
        mesh=mesh,
        scratch_types=[
            pltpu.VMEM_SHARED((_NPAD, _F), jnp.float32),
            pltpu.VMEM((3, _BLK, _CHUNK), jnp.int32),
            pltpu.VMEM((3, _BLK, _CHUNK), jnp.int32),
            pltpu.VMEM((3, _BLK, _CHUNK), jnp.float32),
            pltpu.VMEM((2, _CHUNK, _F), jnp.float32),
            pltpu.SemaphoreType.DMA((3,)),
            pltpu.SemaphoreType.DMA((2,)),
            pltpu.SemaphoreType.DMA((2,)),
        ],
    )


def kernel(x, edge_index, edge_values, W1, b1, W2, b2, W3, b3, Wlin, blin):
    src = edge_index[0]
    dst = edge_index[1]

    pad = _EP - _E
    fill = jnp.arange(pad, dtype=jnp.int32) % _N
    rows = _NW * _NCHUNKS
    srcp = jnp.pad(jnp.concatenate([src, fill]).reshape(rows, _CHUNK),
                   ((0, _BLK), (0, 0)))
    dstp = jnp.pad(jnp.concatenate([dst, fill]).reshape(rows, _CHUNK),
                   ((0, _BLK), (0, 0)))
    evp = jnp.pad(
        jnp.concatenate([edge_values,
                         jnp.zeros((pad,), jnp.float32)]).reshape(rows, _CHUNK),
        ((0, _BLK), (0, 0)))

    edge_pass = _make_edge_pass()

    hw1 = _mm(x, W1)
    p1 = edge_pass(hw1, srcp, dstp, evp)
    x1, hw2 = _fuse(p1, b1.reshape(1, -1), W2)
    p2 = edge_pass(hw2, srcp, dstp, evp)
    x2, hw3 = _fuse(p2, b2.reshape(1, -1), W3)
    p3 = edge_pass(hw3, srcp, dstp, evp)
    out = _final(p3, b3.reshape(1, -1), x1, x2,
                 Wlin[0:_F], Wlin[_F:2 * _F], Wlin[2 * _F:3 * _F],
                 blin.reshape(1, -1))
    return out


# speedup vs baseline: 12.1188x; 1.0814x over previous
import functools

import jax
import jax.numpy as jnp
from jax import lax
from jax.experimental import pallas as pl
from jax.experimental.pallas import tpu as pltpu
from jax.experimental.pallas import tpu_sc as plsc

_N = 10000
_E = 320000
_F = 128
_NCLASS = 40

_NW = 32
_CHUNK = 128
_HALF = 64
_EPW = 10240
_EP = _NW * _EPW
_NCHUNKS = _EPW // _CHUNK
_NPAD = 10240
_RPT = _NPAD // 16

_BN = 1000


def _mm_body(x_ref, w_ref, o_ref):
    o_ref[...] = jnp.dot(x_ref[...], w_ref[...],
                         preferred_element_type=jnp.float32)


def _fuse_body(p_ref, b_ref, w_ref, x_ref, hw_ref):
    xb = jnp.maximum(p_ref[0] + p_ref[1] + b_ref[...], 0.0)
    x_ref[...] = xb
    hw_ref[...] = jnp.dot(xb, w_ref[...], preferred_element_type=jnp.float32)


def _final_body(p_ref, b_ref, x1_ref, x2_ref, wl1_ref, wl2_ref, wl3_ref,
                bl_ref, o_ref):
    x3 = jnp.maximum(p_ref[0] + p_ref[1] + b_ref[...], 0.0)
    logits = (jnp.dot(x1_ref[...], wl1_ref[...],
                      preferred_element_type=jnp.float32)
              + jnp.dot(x2_ref[...], wl2_ref[...],
                        preferred_element_type=jnp.float32)
              + jnp.dot(x3, wl3_ref[...], preferred_element_type=jnp.float32)
              + bl_ref[...])
    m = jnp.max(logits, axis=1, keepdims=True)
    s = logits - m
    o_ref[...] = s - jnp.log(jnp.sum(jnp.exp(s), axis=1, keepdims=True))


def _mm(x, w):
    return pl.pallas_call(
        _mm_body,
        grid=(_N // _BN,),
        in_specs=[pl.BlockSpec((_BN, _F), lambda i: (i, 0)),
                  pl.BlockSpec((_F, _F), lambda i: (0, 0))],
        out_specs=pl.BlockSpec((_BN, _F), lambda i: (i, 0)),
        out_shape=jax.ShapeDtypeStruct((_N, _F), jnp.float32),
    )(x, w)


def _fuse(p, b2d, w):
    return pl.pallas_call(
        _fuse_body,
        grid=(_N // _BN,),
        in_specs=[pl.BlockSpec((2, _BN, _F), lambda i: (0, i, 0)),
                  pl.BlockSpec((1, _F), lambda i: (0, 0)),
                  pl.BlockSpec((_F, _F), lambda i: (0, 0))],
        out_specs=[pl.BlockSpec((_BN, _F), lambda i: (i, 0)),
                   pl.BlockSpec((_BN, _F), lambda i: (i, 0))],
        out_shape=[jax.ShapeDtypeStruct((_N, _F), jnp.float32),
                   jax.ShapeDtypeStruct((_N, _F), jnp.float32)],
    )(p, b2d, w)


def _final(p, b2d, x1, x2, wl1, wl2, wl3, bl2d):
    return pl.pallas_call(
        _final_body,
        grid=(_N // _BN,),
        in_specs=[pl.BlockSpec((2, _BN, _F), lambda i: (0, i, 0)),
                  pl.BlockSpec((1, _F), lambda i: (0, 0)),
                  pl.BlockSpec((_BN, _F), lambda i: (i, 0)),
                  pl.BlockSpec((_BN, _F), lambda i: (i, 0)),
                  pl.BlockSpec((_F, _NCLASS), lambda i: (0, 0)),
                  pl.BlockSpec((_F, _NCLASS), lambda i: (0, 0)),
                  pl.BlockSpec((_F, _NCLASS), lambda i: (0, 0)),
                  pl.BlockSpec((1, _NCLASS), lambda i: (0, 0))],
        out_specs=pl.BlockSpec((_BN, _NCLASS), lambda i: (i, 0)),
        out_shape=jax.ShapeDtypeStruct((_N, _NCLASS), jnp.float32),
    )(p, b2d, x1, x2, wl1, wl2, wl3, bl2d)


_BLK = 8


def _edge_body(hw_hbm, src_hbm, dst_hbm, ev_hbm, out_hbm,
               acc_sh, sblk, dblk, eblk, rows4, sb, sg, ss):
    cid = lax.axis_index("c")
    sid = lax.axis_index("s")
    wid = cid * 16 + sid
    wrow = wid * _NCHUNKS
    nblk = _NCHUNKS // _BLK

    z = jnp.zeros((16,), jnp.float32)

    def zrow(r, carry):
        for j in range(_F // 16):
            rows4[0, r, pl.ds(j * 16, 16)] = z
        return carry

    lax.fori_loop(0, _HALF, zrow, 0)
    for q in range(_RPT // _HALF):
        pltpu.sync_copy(
            rows4.at[0], acc_sh.at[pl.ds(sid * _RPT + q * _HALF, _HALF)])
    plsc.subcore_barrier()

    def blk_copies(to):
        rs = pl.ds(wrow + to * _BLK, _BLK)
        bslot = lax.rem(to, 3)
        sem = sb.at[bslot]
        return (pltpu.make_async_copy(src_hbm.at[rs], sblk.at[bslot], sem),
                pltpu.make_async_copy(dst_hbm.at[rs], dblk.at[bslot], sem),
                pltpu.make_async_copy(ev_hbm.at[rs], eblk.at[bslot], sem))

    def g_start(t, hf, slot):
        bslot = lax.rem(lax.div(t, _BLK), 3)
        pltpu.async_copy(
            hw_hbm.at[sblk.at[bslot, lax.rem(t, _BLK),
                              pl.ds(hf * _HALF, _HALF)]],
            rows4.at[slot], sg.at[slot])

    def g_wait(slot):
        pltpu.make_async_copy(hw_hbm.at[sblk.at[0, 0, pl.ds(0, _HALF)]],
                              rows4.at[slot], sg.at[slot]).wait()

    def s_start(slot, bslot, bi, hf):
        pltpu.async_copy(
            rows4.at[slot],
            acc_sh.at[dblk.at[bslot, bi, pl.ds(hf * _HALF, _HALF)]],
            ss.at[slot], add=True)

    def s_wait(slot):
        pltpu.make_async_copy(rows4.at[slot],
                              acc_sh.at[dblk.at[0, 0, pl.ds(0, _HALF)]],
                              ss.at[slot]).wait()

    def scale(slot, bslot, bi, hf):
        def group(g, c2):
            ev16 = eblk[bslot, bi, pl.ds(hf * _HALF + g * 16, 16)]
            for e in range(16):
                w = lax.gather(
                    ev16, jnp.full((16, 1), e, jnp.int32),
                    dimension_numbers=lax.GatherDimensionNumbers(
                        offset_dims=(), collapsed_slice_dims=(0,),
                        start_index_map=(0,)),
                    slice_sizes=(1,),
                    mode=lax.GatherScatterMode.PROMISE_IN_BOUNDS)
                k = g * 16 + e
                for j in range(_F // 16):
                    rows4[slot, k, pl.ds(j * 16, 16)] = (
                        rows4[slot, k, pl.ds(j * 16, 16)] * w)
            return c2

        lax.fori_loop(0, _HALF // 16, group, 0)

    for c in blk_copies(0):
        c.start()
    for c in blk_copies(0):
        c.wait()
    g_start(0, 0, 0)
    g_start(0, 1, 1)

    def outer(to, carry):
        bslot = lax.rem(to, 3)

        @pl.when(to < nblk - 1)
        def _():
            for c in blk_copies(to + 1):
                c.start()

        def inner(ti, c2):
            for u in (0, 1):
                t = to * _BLK + 2 * ti + u
                bi = 2 * ti + u
                for hf in (0, 1):
                    q = 2 * u + hf
                    qn = (q + 2) % 4
                    g_wait(q)

                    @pl.when(t + 1 < _NCHUNKS)
                    def _():
                        @pl.when(t >= 1)
                        def _():
                            s_wait(qn)
                        if u == 1 and hf == 0:
                            @pl.when(ti == _BLK // 2 - 1)
                            def _():
                                for c in blk_copies(to + 1):
                                    c.wait()
                        g_start(t + 1, hf, qn)
                    scale(q, bslot, bi, hf)
                    s_start(q, bslot, bi, hf)
            return c2

        lax.fori_loop(0, _BLK // 2, inner, 0)
        return carry

    lax.fori_loop(0, nblk, outer, 0)
    s_wait(0)
    s_wait(1)
    s_wait(2)
    s_wait(3)
    plsc.subcore_barrier()
    pltpu.sync_copy(acc_sh.at[pl.ds(sid * _RPT, _RPT)],
                    out_hbm.at[cid, pl.ds(sid * _RPT, _RPT)])


@functools.cache
def _make_edge_pass():
    mesh = plsc.VectorSubcoreMesh(core_axis_name="c", subcore_axis_name="s")
    return pl.kernel(
        _edge_body,
        out_type=jax.ShapeDtypeStruct((2, _NPAD, _F), jnp.float32),
        mesh=mesh,
        scratch_types=[
            pltpu.VMEM_SHARED((_NPAD, _F), jnp.float32),
            pltpu.VMEM((3, _BLK, _CHUNK), jnp.int32),
            pltpu.VMEM((3, _BLK, _CHUNK), jnp.int32),
            pltpu.VMEM((3, _BLK, _CHUNK), jnp.float32),
            pltpu.VMEM((4, _HALF, _F), jnp.float32),
            pltpu.SemaphoreType.DMA((3,)),
            pltpu.SemaphoreType.DMA((4,)),
            pltpu.SemaphoreType.DMA((4,)),
        ],
    )


def kernel(x, edge_index, edge_values, W1, b1, W2, b2, W3, b3, Wlin, blin):
    src = edge_index[0]
    dst = edge_index[1]

    pad = _EP - _E
    fill = jnp.arange(pad, dtype=jnp.int32) % _N
    rows = _NW * _NCHUNKS
    srcp = jnp.pad(jnp.concatenate([src, fill]).reshape(rows, _CHUNK),
                   ((0, _BLK), (0, 0)))
    dstp = jnp.pad(jnp.concatenate([dst, fill]).reshape(rows, _CHUNK),
                   ((0, _BLK), (0, 0)))
    evp = jnp.pad(
        jnp.concatenate([edge_values,
                         jnp.zeros((pad,), jnp.float32)]).reshape(rows, _CHUNK),
        ((0, _BLK), (0, 0)))

    edge_pass = _make_edge_pass()

    hw1 = _mm(x, W1)
    p1 = edge_pass(hw1, srcp, dstp, evp)
    x1, hw2 = _fuse(p1, b1.reshape(1, -1), W2)
    p2 = edge_pass(hw2, srcp, dstp, evp)
    x2, hw3 = _fuse(p2, b2.reshape(1, -1), W3)
    p3 = edge_pass(hw3, srcp, dstp, evp)
    out = _final(p3, b3.reshape(1, -1), x1, x2,
                 Wlin[0:_F], Wlin[_F:2 * _F], Wlin[2 * _F:3 * _F],
                 blin.reshape(1, -1))
    return out
